# baseline (speedup 1.0000x reference)
import jax
import jax.numpy as jnp
from jax import lax
from jax.experimental import pallas as pl
from jax.experimental.pallas import tpu as pltpu
from jax.experimental.pallas import tpu_sc as plsc

N_NODES = 50000
HIDDEN = 256
SEGS = 512
OUT_DIM = 128

CHUNK = 80
NCHUNKS = N_NODES // CHUNK
NRR = 16
NC = 2
HH = HIDDEN // NC
NCG = HH // 16
CNT_W = 16
ACC = SEGS * HH
CNT = SEGS * CNT_W


SLICE = ACC // NRR
NPH = 8
NBUF = 4
MAXROWS = 3200


def _sc_segment_sum(x_hbm, batch_hbm, zsum_hbm, zcnt_hbm,
                    sums_out, cnts_out,
                    accf, cntf, xbuf, idx_all, spl, stage, rbuf,
                    semx, semr):
    cid = lax.axis_index("c")
    sid = lax.axis_index("s")
    rr = sid
    colbase = cid * HH

    pltpu.sync_copy(zsum_hbm, accf)
    pltpu.sync_copy(zcnt_hbm, cntf)

    start = rr * NCHUNKS // NRR
    end = (rr + 1) * NCHUNKS // NRR

    import jax as _jax
    with _jax.named_scope("sc_init"):
        pltpu.sync_copy(batch_hbm.at[pl.ds(start * CHUNK, MAXROWS)], idx_all)

    def issue(ci, buf):
        base = ci * CHUNK
        pltpu.async_copy(x_hbm.at[pl.ds(base, CHUNK), pl.ds(colbase, HH)],
                         xbuf.at[buf], semx)

    def drain(buf):
        pltpu.make_async_copy(x_hbm.at[pl.ds(0, CHUNK), pl.ds(0, HH)],
                              xbuf.at[buf], semx).wait()

    for j in range(NBUF - 1):
        issue(jnp.minimum(start + j, end - 1), j)

    one16 = jnp.ones((16,), jnp.float32)
    sixteen16 = jnp.full((16,), 16.0, jnp.float32)

    def chunk(ci, carry):
        k = ci - start
        par = lax.rem(k, NBUF)
        drain(par)
        issue(jnp.minimum(ci + NBUF - 1, end - 1), lax.rem(k + NBUF - 1, NBUF))
        ib = k * CHUNK

        def group(g, carry2):
            idx_grp = idx_all[pl.ds(ib + g * 16, 16)]
            s0 = idx_grp[0]
            s15 = idx_grp[15]
            r0 = g * 16

            @pl.when(s0 == s15)
            def _fast():
                acc = [xbuf[par, r0, pl.ds(cg * 16, 16)] for cg in range(NCG)]
                for lane in range(1, 16):
                    for cg in range(NCG):
                        acc[cg] = acc[cg] + xbuf[par, r0 + lane,
                                                 pl.ds(cg * 16, 16)]
                sb = s0 * HH
                for cg in range(NCG):
                    o = sb + cg * 16
                    accf[pl.ds(o, 16)] = accf[pl.ds(o, 16)] + acc[cg]
                cb = s0 * CNT_W
                cntf[pl.ds(cb, 16)] = cntf[pl.ds(cb, 16)] + sixteen16

            @pl.when(s0 != s15)
            def _slow():
                for lane in range(16):
                    s = idx_grp[lane]
                    sb = s * HH
                    for cg in range(NCG):
                        o = sb + cg * 16
                        accf[pl.ds(o, 16)] = (accf[pl.ds(o, 16)]
                                              + xbuf[par, r0 + lane,
                                                     pl.ds(cg * 16, 16)])
                    cb = s * CNT_W
                    cntf[pl.ds(cb, 16)] = cntf[pl.ds(cb, 16)] + one16

            return carry2
        lax.fori_loop(0, CHUNK // 16, group, 0)
        return carry
    with jax.named_scope("sc_mainloop"):
        lax.fori_loop(start, end, chunk, 0)

        for j in range(NBUF - 1):
            drain(lax.rem(end - start + j, NBUF))

    HACC = ACC // NPH
    HSLICE = SLICE // NPH
    jax2 = jax
    for ph in range(NPH):
      with jax2.named_scope(f"sc_reduce{ph}"):
        pltpu.sync_copy(accf.at[pl.ds(ph * HACC, HACC)],
                        spl.at[pl.ds(sid * HACC, HACC)])
        plsc.subcore_barrier()
        off = sid * HSLICE
        for rnd in range(4):
            for j in range(4):
                p = rnd * 4 + j
                pltpu.async_copy(spl.at[pl.ds(p * HACC + off, HSLICE)],
                                 stage.at[j], semr)
            for j in range(4):
                pltpu.make_async_copy(spl.at[pl.ds(0, HSLICE)],
                                      stage.at[j], semr).wait()

            def red(g, carry, _rnd=rnd):
                v = (stage[0, pl.ds(g * 16, 16)] + stage[1, pl.ds(g * 16, 16)]
                     + stage[2, pl.ds(g * 16, 16)]
                     + stage[3, pl.ds(g * 16, 16)])
                if _rnd == 0:
                    rbuf[pl.ds(g * 16, 16)] = v
                else:
                    rbuf[pl.ds(g * 16, 16)] = rbuf[pl.ds(g * 16, 16)] + v
                return carry
            lax.fori_loop(0, HSLICE // 16, red, 0)
        pltpu.sync_copy(
            rbuf, sums_out.at[pl.ds(cid * ACC + ph * HACC + off, HSLICE)])
        plsc.subcore_barrier()

    pltpu.sync_copy(cntf, cnts_out.at[rr, cid])


def _finish_kernel(sums_ref, cnts_ref, w_ref, b_ref, out_ref):
    c = jnp.sum(cnts_ref[...], axis=0)[0][:, 0:1]
    inv = 1.0 / jnp.clip(c, 1.0, None)
    m0 = sums_ref[0] * inv
    m1 = sums_ref[1] * inv
    dn = (((1,), (1,)), ((), ()))
    out = (lax.dot_general(m0, w_ref[:, 0:HH], dimension_numbers=dn,
                           preferred_element_type=jnp.float32)
           + lax.dot_general(m1, w_ref[:, HH:HIDDEN], dimension_numbers=dn,
                             preferred_element_type=jnp.float32))
    out_ref[...] = out + b_ref[...]


def kernel(x, batch, W, b):
    batch_i32 = batch.astype(jnp.int32)
    zsum = jnp.zeros((ACC,), jnp.float32)
    zcnt = jnp.zeros((CNT,), jnp.float32)

    mesh = plsc.VectorSubcoreMesh(core_axis_name="c", subcore_axis_name="s")
    sc_call = pl.kernel(
        _sc_segment_sum,
        mesh=mesh,
        out_type=[
            jax.ShapeDtypeStruct((NC * ACC,), jnp.float32),
            jax.ShapeDtypeStruct((NRR, NC, CNT), jnp.float32),
        ],
        scratch_types=[
            pltpu.VMEM((ACC,), jnp.float32),
            pltpu.VMEM((CNT,), jnp.float32),
            pltpu.VMEM((NBUF, CHUNK, HH), jnp.float32),
            pltpu.VMEM((MAXROWS,), jnp.int32),
            pltpu.VMEM_SHARED((NRR * ACC // NPH,), jnp.float32),
            pltpu.VMEM((4, SLICE // NPH), jnp.float32),
            pltpu.VMEM((SLICE // NPH,), jnp.float32),
            pltpu.SemaphoreType.DMA,
            pltpu.SemaphoreType.DMA,
        ],
    )
    sums, cnts = sc_call(x, batch_i32, zsum, zcnt)
    sums = sums.reshape(NC, SEGS, HH)
    cnts = cnts.reshape(NRR, NC, SEGS, CNT_W)

    out = pl.pallas_call(
        _finish_kernel,
        out_shape=jax.ShapeDtypeStruct((SEGS, OUT_DIM), jnp.float32),
    )(sums, cnts, W, b.reshape(1, OUT_DIM))
    return out


# speedup vs baseline: 1.1069x; 1.1069x over previous
import jax
import jax.numpy as jnp
from jax import lax
from jax.experimental import pallas as pl
from jax.experimental.pallas import tpu as pltpu
from jax.experimental.pallas import tpu_sc as plsc

N_NODES = 50000
HIDDEN = 256
SEGS = 512
OUT_DIM = 128

CHUNK = 80
NCHUNKS = N_NODES // CHUNK
NRR = 16
NC = 2
HH = HIDDEN // NC
NCG = HH // 16
CNT_W = 16
ACC = SEGS * HH
CNT = SEGS * CNT_W


SLICE = ACC // NRR
CSLICE = CNT // NRR
NPH = 8
NBUF = 3
MAXROWS = 3200


def _sc_segment_sum(x_hbm, batch_hbm, zsum_hbm, zcnt_hbm,
                    sums_out, cnts_out,
                    accf, cntf, xbuf, idx_all, spl, stage, rbuf,
                    semx, semr):
    cid = lax.axis_index("c")
    sid = lax.axis_index("s")
    rr = sid
    colbase = cid * HH

    pltpu.sync_copy(zsum_hbm, accf)
    pltpu.sync_copy(zcnt_hbm, cntf)

    start = rr * NCHUNKS // NRR
    end = (rr + 1) * NCHUNKS // NRR

    pltpu.sync_copy(batch_hbm.at[pl.ds(start * CHUNK, MAXROWS)], idx_all)

    def issue(ci, buf):
        base = ci * CHUNK
        pltpu.async_copy(x_hbm.at[pl.ds(base, CHUNK), pl.ds(colbase, HH)],
                         xbuf.at[buf], semx)

    def drain(buf):
        pltpu.make_async_copy(x_hbm.at[pl.ds(0, CHUNK), pl.ds(0, HH)],
                              xbuf.at[buf], semx).wait()

    for j in range(NBUF - 1):
        issue(jnp.minimum(start + j, end - 1), j)

    one16 = jnp.ones((16,), jnp.float32)
    sixteen16 = jnp.full((16,), 16.0, jnp.float32)

    def chunk(ci, carry):
        k = ci - start
        par = lax.rem(k, NBUF)
        drain(par)
        issue(jnp.minimum(ci + NBUF - 1, end - 1), lax.rem(k + NBUF - 1, NBUF))
        ib = k * CHUNK

        def group(g, carry2):
            idx_grp = idx_all[pl.ds(ib + g * 16, 16)]
            s0 = idx_grp[0]
            s15 = idx_grp[15]
            r0 = g * 16

            @pl.when(s0 == s15)
            def _fast():
                acc = [xbuf[par, r0, pl.ds(cg * 16, 16)] for cg in range(NCG)]
                for lane in range(1, 16):
                    for cg in range(NCG):
                        acc[cg] = acc[cg] + xbuf[par, r0 + lane,
                                                 pl.ds(cg * 16, 16)]
                sb = s0 * HH
                for cg in range(NCG):
                    o = sb + cg * 16
                    accf[pl.ds(o, 16)] = accf[pl.ds(o, 16)] + acc[cg]
                cb = s0 * CNT_W
                cntf[pl.ds(cb, 16)] = cntf[pl.ds(cb, 16)] + sixteen16

            @pl.when(s0 != s15)
            def _slow():
                for lane in range(16):
                    s = idx_grp[lane]
                    sb = s * HH
                    for cg in range(NCG):
                        o = sb + cg * 16
                        accf[pl.ds(o, 16)] = (accf[pl.ds(o, 16)]
                                              + xbuf[par, r0 + lane,
                                                     pl.ds(cg * 16, 16)])
                    cb = s * CNT_W
                    cntf[pl.ds(cb, 16)] = cntf[pl.ds(cb, 16)] + one16

            return carry2
        lax.fori_loop(0, CHUNK // 16, group, 0)
        return carry
    with jax.named_scope("sc_mainloop"):
        lax.fori_loop(start, end, chunk, 0)

        for j in range(NBUF - 1):
            drain(lax.rem(end - start + j, NBUF))

    HACC = ACC // NPH
    HSLICE = SLICE // NPH
    for ph in range(NPH):
      with jax.named_scope(f"sc_reduce{ph}"):
        pltpu.sync_copy(accf.at[pl.ds(ph * HACC, HACC)],
                        spl.at[pl.ds(sid * HACC, HACC)])
        plsc.subcore_barrier()
        off = sid * HSLICE
        for p in range(NRR):
            pltpu.async_copy(spl.at[pl.ds(p * HACC + off, HSLICE)],
                             stage.at[p], semr)
        for p in range(NRR):
            pltpu.make_async_copy(spl.at[pl.ds(0, HSLICE)],
                                  stage.at[p], semr).wait()

        def red(g, carry):
            v = stage[0, pl.ds(g * 16, 16)]
            for p in range(1, NRR):
                v = v + stage[p, pl.ds(g * 16, 16)]
            rbuf[pl.ds(g * 16, 16)] = v
            return carry
        lax.fori_loop(0, HSLICE // 16, red, 0)
        pltpu.sync_copy(
            rbuf, sums_out.at[pl.ds(cid * ACC + ph * HACC + off, HSLICE)])
        plsc.subcore_barrier()

    with jax.named_scope("sc_reduce_cnt"):
        pltpu.sync_copy(cntf, spl.at[pl.ds(sid * CNT, CNT)])
        plsc.subcore_barrier()
        coff = sid * CSLICE
        for p in range(NRR):
            pltpu.async_copy(spl.at[pl.ds(p * CNT + coff, CSLICE)],
                             stage.at[p], semr)
        for p in range(NRR):
            pltpu.make_async_copy(spl.at[pl.ds(0, CSLICE)],
                                  stage.at[p], semr).wait()

        def cred(g, carry):
            v = stage[0, pl.ds(g * 16, 16)]
            for p in range(1, NRR):
                v = v + stage[p, pl.ds(g * 16, 16)]
            rbuf[pl.ds(g * 16, 16)] = v
            return carry
        lax.fori_loop(0, CSLICE // 16, cred, 0)
        pltpu.sync_copy(rbuf.at[pl.ds(0, CSLICE)],
                        cnts_out.at[pl.ds(cid * CNT + coff, CSLICE)])


def _finish_kernel(sums_ref, cnts_ref, w_ref, b_ref, out_ref):
    c = cnts_ref[0][:, 0:1]
    inv = 1.0 / jnp.clip(c, 1.0, None)
    m0 = sums_ref[0] * inv
    m1 = sums_ref[1] * inv
    dn = (((1,), (1,)), ((), ()))
    out = (lax.dot_general(m0, w_ref[:, 0:HH], dimension_numbers=dn,
                           preferred_element_type=jnp.float32)
           + lax.dot_general(m1, w_ref[:, HH:HIDDEN], dimension_numbers=dn,
                             preferred_element_type=jnp.float32))
    out_ref[...] = out + b_ref[...]


def kernel(x, batch, W, b):
    batch_i32 = batch.astype(jnp.int32)
    zsum = jnp.zeros((ACC,), jnp.float32)
    zcnt = jnp.zeros((CNT,), jnp.float32)

    mesh = plsc.VectorSubcoreMesh(core_axis_name="c", subcore_axis_name="s")
    sc_call = pl.kernel(
        _sc_segment_sum,
        mesh=mesh,
        out_type=[
            jax.ShapeDtypeStruct((NC * ACC,), jnp.float32),
            jax.ShapeDtypeStruct((NC * CNT,), jnp.float32),
        ],
        scratch_types=[
            pltpu.VMEM((ACC,), jnp.float32),
            pltpu.VMEM((CNT,), jnp.float32),
            pltpu.VMEM((NBUF, CHUNK, HH), jnp.float32),
            pltpu.VMEM((MAXROWS,), jnp.int32),
            pltpu.VMEM_SHARED((NRR * ACC // NPH,), jnp.float32),
            pltpu.VMEM((NRR, SLICE // NPH), jnp.float32),
            pltpu.VMEM((SLICE // NPH,), jnp.float32),
            pltpu.SemaphoreType.DMA,
            pltpu.SemaphoreType.DMA,
        ],
    )
    sums, cnts = sc_call(x, batch_i32, zsum, zcnt)
    sums = sums.reshape(NC, SEGS, HH)
    cnts = cnts.reshape(NC, SEGS, CNT_W)

    out = pl.pallas_call(
        _finish_kernel,
        out_shape=jax.ShapeDtypeStruct((SEGS, OUT_DIM), jnp.float32),
    )(sums, cnts, W, b.reshape(1, OUT_DIM))
    return out
